# baseline (speedup 1.0000x reference)
import functools

import jax
import jax.numpy as jnp
from jax import lax
from jax.experimental import pallas as pl
from jax.experimental.pallas import tpu as pltpu
from jax.experimental.pallas import tpu_sc as plsc

N = 10000
E = 320000
H = 128
L = 7
G = 128
HC = H // 2
NC = 2
NS = 16
CH = 80
R8 = 624
RT = N - NS * R8
EPS16 = 1e-16

_mesh = plsc.VectorSubcoreMesh(core_axis_name="c", subcore_axis_name="s")


def _zero_rows(zeros_hbm, acc, s):
    pltpu.sync_copy(zeros_hbm.at[pl.ds(0, R8)], acc.at[pl.ds(s * R8, R8)])

    @pl.when(s == NS - 1)
    def _tail():
        pltpu.sync_copy(zeros_hbm.at[pl.ds(0, RT)], acc.at[pl.ds(NS * R8, RT)])


def _out_rows(acc, out, c, s):
    pltpu.sync_copy(acc.at[pl.ds(s * R8, R8)], out.at[c, pl.ds(s * R8, R8)])

    @pl.when(s == NS - 1)
    def _tail():
        pltpu.sync_copy(acc.at[pl.ds(NS * R8, RT)], out.at[c, pl.ds(NS * R8, RT)])


@functools.partial(
    pl.kernel,
    out_type=(jax.ShapeDtypeStruct((2, N, H), jnp.float32),
              jax.ShapeDtypeStruct((N, H), jnp.float32)),
    mesh=_mesh,
    scratch_types=[
        pltpu.VMEM_SHARED((N, H), jnp.float32),
        pltpu.VMEM((CH,), jnp.int32),
        pltpu.VMEM((CH, H), jnp.float32),
        pltpu.VMEM((CH,), jnp.int32),
        pltpu.VMEM((CH, H), jnp.float32),
    ],
)
def _sc_prep(srcdst_hbm, x_hbm, atom_hbm, ones_hbm, zeros_hbm,
             deg_out, h0_out,
             acc, idx_e, ones_v, xb_v, arows_v):
    c = lax.axis_index("c")
    s = lax.axis_index("s")
    _zero_rows(zeros_hbm, acc, s)
    pltpu.sync_copy(ones_hbm, ones_v)
    plsc.subcore_barrier()
    w = c * NS + s

    def deg_body(k, carry):
        base = c * E + s * (E // NS) + k * CH
        pltpu.sync_copy(srcdst_hbm.at[pl.ds(base, CH)], idx_e)
        pltpu.sync_copy(ones_v, acc.at[idx_e], add=True)
        return carry

    lax.fori_loop(0, E // NS // CH, deg_body, 0)

    n_chunks = N // CH

    def h0_body(j, carry):
        k = w + j * (NC * NS)

        @pl.when(k < n_chunks)
        def _do():
            pltpu.sync_copy(x_hbm.at[pl.ds(k * CH, CH)], xb_v)
            pltpu.sync_copy(atom_hbm.at[xb_v], arows_v)
            pltpu.sync_copy(arows_v, h0_out.at[pl.ds(k * CH, CH)])

        return carry

    lax.fori_loop(0, (n_chunks + NC * NS - 1) // (NC * NS), h0_body, 0)
    plsc.subcore_barrier()
    _out_rows(acc, deg_out, c, s)


@functools.partial(
    pl.kernel,
    out_type=jax.ShapeDtypeStruct((2, N, H), jnp.float32),
    mesh=_mesh,
    scratch_types=[
        pltpu.VMEM_SHARED((N, H), jnp.float32),
        pltpu.VMEM((CH,), jnp.int32),
        pltpu.VMEM((CH,), jnp.int32),
        pltpu.VMEM((CH, H), jnp.float32),
    ],
)
def _sc_spmm(g_hbm, src_hbm, dst_hbm, zeros_hbm, acc_out,
             acc, idx_s, idx_d, grows):
    c = lax.axis_index("c")
    s = lax.axis_index("s")
    _zero_rows(zeros_hbm, acc, s)
    plsc.subcore_barrier()
    w = c * NS + s

    def body(k, carry):
        base = w * (E // (NC * NS)) + k * CH
        pltpu.sync_copy(src_hbm.at[pl.ds(base, CH)], idx_s)
        pltpu.sync_copy(dst_hbm.at[pl.ds(base, CH)], idx_d)
        pltpu.sync_copy(g_hbm.at[idx_s], grows)
        pltpu.sync_copy(grows, acc.at[idx_d], add=True)
        return carry

    lax.fori_loop(0, E // (NC * NS) // CH, body, 0)
    plsc.subcore_barrier()
    _out_rows(acc, acc_out, c, s)


@functools.partial(
    pl.kernel,
    out_type=jax.ShapeDtypeStruct((2, N, H), jnp.float32),
    mesh=_mesh,
    scratch_types=[
        pltpu.VMEM_SHARED((N, H), jnp.float32),
        pltpu.VMEM((CH,), jnp.int32),
        pltpu.VMEM((CH,), jnp.int32),
        pltpu.VMEM((CH,), jnp.int32),
        pltpu.VMEM((CH, H), jnp.float32),
        pltpu.VMEM((CH, H), jnp.float32),
        pltpu.VMEM((CH, H), jnp.float32),
    ],
)
def _sc_genconv(hh_hbm, src_hbm, dst_hbm, attr_hbm, bond_hbm, zeros_hbm,
                zzm_out,
                acc, idx_s, idx_d, idx_a, hrows, brows, zzm):
    c = lax.axis_index("c")
    s = lax.axis_index("s")
    _zero_rows(zeros_hbm, acc, s)
    plsc.subcore_barrier()
    coff = c * HC

    def body(k, carry):
        base = s * (E // NS) + k * CH
        pltpu.sync_copy(src_hbm.at[pl.ds(base, CH)], idx_s)
        pltpu.sync_copy(dst_hbm.at[pl.ds(base, CH)], idx_d)
        pltpu.sync_copy(attr_hbm.at[pl.ds(base, CH)], idx_a)
        pltpu.sync_copy(hh_hbm.at[idx_s], hrows)
        pltpu.sync_copy(bond_hbm.at[idx_a], brows)

        def ebody(e, carry2):
            for f in range(HC // 16):
                hv = hrows[e, pl.ds(coff + f * 16, 16)]
                bv = brows[e, pl.ds(coff + f * 16, 16)]
                m = jnp.maximum(hv + bv, 0.0) + 1e-7
                z = jnp.exp(m)
                zzm[e, pl.ds(f * 16, 16)] = z
                zzm[e, pl.ds(HC + f * 16, 16)] = z * m
            return carry2

        lax.fori_loop(0, CH, ebody, 0)
        pltpu.sync_copy(zzm, acc.at[idx_d], add=True)
        return carry

    lax.fori_loop(0, E // NS // CH, body, 0)
    plsc.subcore_barrier()
    _out_rows(acc, zzm_out, c, s)


BN = 2000


def _deg_inv(deg_ref):
    deg_s = deg_ref[0, :, 0:1] + 1.0
    deg_d = deg_ref[1, :, 0:1] + 1.0
    return lax.rsqrt(deg_s), lax.rsqrt(deg_d)


_full_nh = pl.BlockSpec((BN, H), lambda i: (i, 0))
_pair_spec = pl.BlockSpec((2, BN, H), lambda i: (0, i, 0))


def _g_body(h0_ref, deg_ref, g_ref):
    inv_s, _ = _deg_inv(deg_ref)
    g_ref[...] = h0_ref[...] * inv_s


_g_call = pl.pallas_call(
    _g_body,
    grid=(N // BN,),
    in_specs=[_full_nh, _pair_spec],
    out_specs=_full_nh,
    out_shape=jax.ShapeDtypeStruct((N, H), jnp.float32),
)


def _node0_body(h0_ref, accp_ref, deg_ref, h_ref):
    inv_s, inv_d = _deg_inv(deg_ref)
    h0 = h0_ref[...]
    ah = inv_d * (accp_ref[0] + accp_ref[1]) + (inv_s * inv_d) * h0
    h_ref[...] = (h0 + ah) * 0.5


_node0_call = pl.pallas_call(
    _node0_body,
    grid=(N // BN,),
    in_specs=[_full_nh, _pair_spec, _pair_spec],
    out_specs=_full_nh,
    out_shape=jax.ShapeDtypeStruct((N, H), jnp.float32),
)


def _layer_norm_tc(v, g_row, b_row):
    mu = jnp.mean(v, axis=-1, keepdims=True)
    var = jnp.mean((v - mu) * (v - mu), axis=-1, keepdims=True)
    return (v - mu) / jnp.sqrt(var + 1e-5) * g_row + b_row


def _act_tc(layer, v):
    if layer in (1, 5):
        return jax.nn.sigmoid(v)
    if layer in (2, 6):
        return jnp.clip(jnp.maximum(v, 0.0), 0.0, 6.0)
    if layer == 4:
        return jnp.tanh(v)
    return jnp.maximum(v, 0.0)


def _make_mlp(l):
    has_res = l >= 1
    last = l == L - 1

    def body(*refs):
        i = 0
        hh_ref = refs[i]; i += 1
        if has_res:
            hp_ref = refs[i]; i += 1
        zzm_ref = refs[i]; i += 1
        w1_ref = refs[i]; i += 1
        b1_ref = refs[i]; i += 1
        w2_ref = refs[i]; i += 1
        b2_ref = refs[i]; i += 1
        g_ref = refs[i]; i += 1
        bt_ref = refs[i]; i += 1
        outs = refs[i:]

        z = jnp.concatenate([zzm_ref[0, :, :HC], zzm_ref[1, :, :HC]], axis=-1)
        zm = jnp.concatenate([zzm_ref[0, :, HC:], zzm_ref[1, :, HC:]], axis=-1)
        agg = zm / (z + EPS16)
        outp = hh_ref[...] + agg
        t = jnp.maximum(
            jnp.dot(outp, w1_ref[...], preferred_element_type=jnp.float32,
                    precision=lax.Precision.HIGHEST)
            + b1_ref[...], 0.0)
        y = jnp.dot(t, w2_ref[...], preferred_element_type=jnp.float32,
                    precision=lax.Precision.HIGHEST) + b2_ref[...]
        if has_res:
            y = y + hp_ref[...]
        h1 = _layer_norm_tc(y, g_ref[...], bt_ref[...])
        if last:
            outs[0][...] = h1
        else:
            outs[0][...] = y
            outs[1][...] = _act_tc(l + 1, h1)

    def wspec(shape):
        return pl.BlockSpec(shape, lambda i, _s=shape: tuple(0 for _ in _s))

    in_specs = [_full_nh]
    if has_res:
        in_specs.append(_full_nh)
    in_specs += [_pair_spec,
                 wspec((H, 2 * H)), wspec((1, 2 * H)),
                 wspec((2 * H, H)), wspec((1, H)),
                 wspec((1, H)), wspec((1, H))]
    if last:
        out_specs = _full_nh
        out_shape = jax.ShapeDtypeStruct((N, H), jnp.float32)
    else:
        out_specs = [_full_nh, _full_nh]
        out_shape = [jax.ShapeDtypeStruct((N, H), jnp.float32),
                     jax.ShapeDtypeStruct((N, H), jnp.float32)]
    return pl.pallas_call(
        body, grid=(N // BN,),
        in_specs=in_specs, out_specs=out_specs, out_shape=out_shape)


_mlp_calls = [_make_mlp(l) for l in range(L)]


def _pool_body(h_ref, batch_ref, pw_ref, pb_ref, out_ref):
    b_row = batch_ref[...]
    seg = lax.broadcasted_iota(jnp.int32, (G, N), 0)
    p = (seg == jnp.broadcast_to(b_row, (G, N))).astype(jnp.float32)
    ssum = jnp.dot(p, h_ref[...], preferred_element_type=jnp.float32,
                   precision=lax.Precision.HIGHEST)
    cnt = jnp.sum(p, axis=1, keepdims=True)
    pooled = ssum / (cnt + EPS16)
    out_ref[...] = jnp.sum(pooled * pw_ref[...], axis=1, keepdims=True) + pb_ref[...]


_pool_call = pl.pallas_call(
    _pool_body,
    out_shape=jax.ShapeDtypeStruct((G, 1), jnp.float32),
)


def kernel(atom_emb, bond_emb, W1s, b1s, W2s, b2s, ln_g, ln_b, pred_W, pred_b,
           x, edge_index, edge_attr, batch):
    f32 = jnp.float32
    i32 = jnp.int32
    src = edge_index[0].astype(i32)
    dst = edge_index[1].astype(i32)
    attr = edge_attr.astype(i32)
    xi = x.astype(i32)

    atom_sc = atom_emb * 0.8
    zeros128 = jnp.zeros((R8, H), f32)
    ones128 = jnp.ones((CH, H), f32)
    srcdst = jnp.concatenate([src, dst])

    deg, h0 = _sc_prep(srcdst, xi, atom_sc, ones128, zeros128)
    g = _g_call(h0, deg)
    accp = _sc_spmm(g, src, dst, zeros128)
    h = _node0_call(h0, accp, deg)

    hh = h
    hprev = None
    for l in range(L):
        zzm = _sc_genconv(hh, src, dst, attr, bond_emb, zeros128)
        args = [hh] + ([hprev] if l >= 1 else []) + [
            zzm, W1s[l], b1s[l].reshape(1, 2 * H), W2s[l],
            b2s[l].reshape(1, H), ln_g[l].reshape(1, H), ln_b[l].reshape(1, H)]
        if l == L - 1:
            h_fin = _mlp_calls[l](*args)
        else:
            hprev, hh = _mlp_calls[l](*args)

    return _pool_call(h_fin, batch.astype(i32).reshape(1, N),
                      pred_W.reshape(1, H), pred_b.reshape(1, 1))
